# Initial kernel scaffold; baseline (speedup 1.0000x reference)
import jax
import jax.numpy as jnp
from jax.experimental import pallas as pl


def kernel(z, pos, batch, tda_feat, emb, mlp_w1, mlp_b1, mlp_w2, mlp_b2, cf1_w, cf2_w, cf2_b, lin_w, lin_b, lin1_w, lin1_b, lin2_w, lin2_b, tda_w1, tda_b1, tda_w2, tda_b2, fin_w, fin_b):
    raise NotImplementedError("write your pallas kernel here")


# speedup vs baseline: 5.9761x; 5.9761x over previous
import functools

import jax
import jax.numpy as jnp
import numpy as np
from jax.experimental import pallas as pl

N = 50000
MOL = 25
G = N // MOL
HID = 128
NF = 128
NG = 50
NI = 6
CUTOFF = 5.0
TDA = 2
LOG2 = float(np.log(2.0))
COEFF = -0.5 / float((CUTOFF / (NG - 1)) ** 2)

AP = 32
PAIR = MOL * MOL
PPAD = 640
B = 8
RB = B * PPAD
NB = B * AP

_i = np.repeat(np.arange(MOL), MOL)
_j = np.tile(np.arange(MOL), MOL)
_off_diag = (_i != _j)

_A = np.zeros((PPAD, AP), np.float32)
_A[np.arange(PAIR), _i] += 1.0
_A[np.arange(PAIR), _j] -= 1.0

_GM = np.zeros((PPAD, AP), np.float32)
_GM[np.arange(PAIR)[_off_diag], _j[_off_diag]] = 1.0

_AM = np.zeros((AP, PPAD), np.float32)
_AM[_i[_off_diag], np.arange(PAIR)[_off_diag]] = 1.0

_MS = np.zeros((B, NB), np.float32)
for _b in range(B):
    _MS[_b, _b * AP:_b * AP + MOL] = 1.0

_OFFS = np.linspace(0.0, CUTOFF, NG, dtype=np.float32).reshape(1, NG)


def _ssp(x):
    return jax.nn.softplus(x) - LOG2


def _body(pos_ref, z_ref, tda_ref, emb_ref, offs_ref,
          w1_ref, b1_ref, w2_ref, b2_ref,
          cf1_ref, cf2_ref, cf2b_ref, linw_ref, linb_ref,
          l1w_ref, l1b_ref, l2w_ref, l2b_ref,
          tw1_ref, tb1_ref, tw2_ref, tb2_ref,
          fwg_ref, fwt_ref, fb_ref,
          a_ref, gm_ref, am_ref, ms_ref,
          out_ref):
    f32 = jnp.float32
    dot = functools.partial(jnp.dot, preferred_element_type=f32)

    pos = pos_ref[...]
    amat = a_ref[...]
    d2 = jnp.concatenate(
        [jnp.sum(jnp.square(dot(amat, pos[b * AP:(b + 1) * AP, :])),
                 axis=1, keepdims=True)
         for b in range(B)], axis=0)
    ew = jnp.sqrt(d2 + 1e-12)
    ea = jnp.exp(COEFF * jnp.square(ew - offs_ref[...]))
    cut = 0.5 * (jnp.cos(ew * (jnp.pi / CUTOFF)) + 1.0)
    cut = jnp.where(d2 <= CUTOFF * CUTOFF, cut, 0.0)

    zc = z_ref[...]
    col = jax.lax.broadcasted_iota(jnp.int32, (NB, 100), 1)
    h = dot((col == zc).astype(f32), emb_ref[...])

    gm = gm_ref[...]
    am = am_ref[...]
    for l in range(NI):
        f1 = _ssp(dot(ea, w1_ref[l]) + b1_ref[l])
        wf = (dot(f1, w2_ref[l]) + b2_ref[l]) * cut
        xl = dot(h, cf1_ref[l])
        xlg = jnp.concatenate(
            [dot(gm, xl[b * AP:(b + 1) * AP, :]) for b in range(B)], axis=0)
        msg = wf * xlg
        agg = jnp.concatenate(
            [dot(am, msg[b * PPAD:(b + 1) * PPAD, :]) for b in range(B)],
            axis=0)
        xc = _ssp(dot(agg, cf2_ref[l]) + cf2b_ref[l])
        h = h + dot(xc, linw_ref[l]) + linb_ref[l]

    hr = _ssp(dot(h, l1w_ref[...]) + l1b_ref[...])
    atom = dot(hr, l2w_ref[...]) + l2b_ref[...]
    geom = dot(ms_ref[...], atom)
    t = jnp.maximum(dot(tda_ref[...], tw1_ref[...]) + tb1_ref[...], 0.0)
    topo = dot(t, tw2_ref[...]) + tb2_ref[...]
    out_ref[...] = (geom * fwg_ref[0, 0] + dot(topo, fwt_ref[...])
                    + fb_ref[0, 0])


def kernel(z, pos, batch, tda_feat, emb,
           mlp_w1, mlp_b1, mlp_w2, mlp_b2,
           cf1_w, cf2_w, cf2_b, lin_w, lin_b,
           lin1_w, lin1_b, lin2_w, lin2_b,
           tda_w1, tda_b1, tda_w2, tda_b2, fin_w, fin_b):
    del batch
    g = tda_feat.shape[0]
    f32 = jnp.float32

    pos_p = jnp.pad(pos.reshape(g, MOL, 3),
                    ((0, 0), (0, AP - MOL), (0, 5))).reshape(g * AP, 8)
    z_p = jnp.pad(z.reshape(g, MOL),
                  ((0, 0), (0, AP - MOL))).reshape(g * AP, 1)
    tda_p = jnp.pad(tda_feat, ((0, 0), (0, 8 - TDA)))
    tw1_p = jnp.pad(tda_w1, ((0, 8 - TDA), (0, 0)))

    def whole(a):
        nd = a.ndim
        return pl.BlockSpec(a.shape, lambda i, _n=nd: (0,) * _n)

    operands = [
        pos_p, z_p, tda_p, emb.astype(f32), jnp.asarray(_OFFS),
        mlp_w1, mlp_b1.reshape(NI, 1, NF), mlp_w2, mlp_b2.reshape(NI, 1, NF),
        cf1_w, cf2_w, cf2_b.reshape(NI, 1, HID), lin_w,
        lin_b.reshape(NI, 1, HID),
        lin1_w, lin1_b.reshape(1, HID // 2), lin2_w, lin2_b.reshape(1, 1),
        tw1_p, tda_b1.reshape(1, 16), tda_w2, tda_b2.reshape(1, 16),
        fin_w[0:1, 0:1], fin_w[1:, :], fin_b.reshape(1, 1),
        jnp.asarray(_A), jnp.asarray(_GM), jnp.asarray(_AM), jnp.asarray(_MS),
    ]
    in_specs = [
        pl.BlockSpec((NB, 8), lambda i: (i, 0)),
        pl.BlockSpec((NB, 1), lambda i: (i, 0)),
        pl.BlockSpec((B, 8), lambda i: (i, 0)),
    ] + [whole(a) for a in operands[3:]]

    out = pl.pallas_call(
        _body,
        grid=(g // B,),
        in_specs=in_specs,
        out_specs=pl.BlockSpec((B, 1), lambda i: (i, 0)),
        out_shape=jax.ShapeDtypeStruct((g, 1), f32),
    )(*operands)
    return out
